# Initial kernel scaffold; baseline (speedup 1.0000x reference)
import jax
import jax.numpy as jnp
from jax.experimental import pallas as pl


def kernel(x, phi_coeffs, Phi_coeffs, lambdas, eta):
    raise NotImplementedError("write your pallas kernel here")


# speedup vs baseline: 4493.3633x; 4493.3633x over previous
import functools

import jax
import jax.numpy as jnp
from jax import lax
from jax.experimental import pallas as pl
from jax.experimental.pallas import tpu as pltpu
from jax.experimental.pallas import tpu_sc as plsc

_NW = 32
_CHUNK = 10000
_VEC = _CHUNK // 16

_PHI_N = 200
_PHI2_N = 100
_PHI_PAD = 208
_PHI2_PAD = 112
_HIDDEN = 3


def _sc_body(x_hbm, phi_hbm, big_hbm, par_hbm, out_hbm,
             xbuf, obuf, phib, bigb, parb):
    nchunks = x_hbm.shape[0] // _CHUNK
    maxc = (nchunks + _NW - 1) // _NW
    cidx = lax.axis_index("c")
    sidx = lax.axis_index("s")
    wid = sidx * 2 + cidx

    pltpu.sync_copy(phi_hbm, phib)
    pltpu.sync_copy(big_hbm, bigb)
    pltpu.sync_copy(par_hbm, parb)
    eta_v = parb[pl.ds(0, 16)]
    lam_v = parb[pl.ds(16, 16)]

    def vec_body(i, _):
        v = xbuf[pl.ds(i * 16, 16)]
        acc = jnp.zeros((16,), jnp.float32)
        for q in range(_HIDDEN):
            s = jnp.clip(v + eta_v * float(q), 0.0, 1.0)
            f = s * float(_PHI_N - 1)
            ii = jnp.clip(f.astype(jnp.int32), 0, _PHI_N - 2)
            t = f - ii.astype(jnp.float32)
            c0 = plsc.load_gather(phib, [ii])
            c1 = plsc.load_gather(phib, [ii + 1])
            phi = (1.0 - t) * c0 + t * c1
            inner = lam_v * phi + float(q)
            g = jnp.clip(inner, -3.0, 3.0)
            f2 = (g + 3.0) * (float(_PHI2_N - 1) / 6.0)
            jj = jnp.clip(f2.astype(jnp.int32), 0, _PHI2_N - 2)
            t2 = f2 - jj.astype(jnp.float32)
            d0 = plsc.load_gather(bigb, [jj])
            d1 = plsc.load_gather(bigb, [jj + 1])
            acc = acc + (1.0 - t2) * d0 + t2 * d1
        obuf[pl.ds(i * 16, 16)] = acc
        return _

    def chunk_body(k, _):
        cid = wid + _NW * k

        @pl.when(cid < nchunks)
        def _go():
            off = cid * _CHUNK
            pltpu.sync_copy(x_hbm.at[pl.ds(off, _CHUNK)], xbuf)
            lax.fori_loop(0, _VEC, vec_body, 0)
            pltpu.sync_copy(obuf, out_hbm.at[pl.ds(off, _CHUNK)])

        return _

    lax.fori_loop(0, maxc, chunk_body, 0)


def _make_sc_kernel(n):
    mesh = plsc.VectorSubcoreMesh(core_axis_name="c", subcore_axis_name="s")
    return pl.kernel(
        _sc_body,
        mesh=mesh,
        compiler_params=pltpu.CompilerParams(needs_layout_passes=False),
        out_type=jax.ShapeDtypeStruct((n,), jnp.float32),
        scratch_types=[
            pltpu.VMEM((_CHUNK,), jnp.float32),
            pltpu.VMEM((_CHUNK,), jnp.float32),
            pltpu.VMEM((_PHI_PAD,), jnp.float32),
            pltpu.VMEM((_PHI2_PAD,), jnp.float32),
            pltpu.VMEM((32,), jnp.float32),
        ],
    )


def kernel(x, phi_coeffs, Phi_coeffs, lambdas, eta):
    n = x.shape[0]
    xf = x.reshape(n)
    phi_p = jnp.zeros((_PHI_PAD,), jnp.float32).at[:_PHI_N].set(phi_coeffs)
    big_p = jnp.zeros((_PHI2_PAD,), jnp.float32).at[:_PHI2_N].set(Phi_coeffs)
    par = jnp.concatenate([
        jnp.full((16,), eta, jnp.float32),
        jnp.full((16,), lambdas[0], jnp.float32),
    ])
    out = _make_sc_kernel(n)(xf, phi_p, big_p, par)
    return out.reshape(n, 1)


# speedup vs baseline: 5843.0700x; 1.3004x over previous
import jax
import jax.numpy as jnp
from jax import lax
from jax.experimental import pallas as pl
from jax.experimental.pallas import tpu as pltpu
from jax.experimental.pallas import tpu_sc as plsc

_NW = 32
_CHUNK = 10000
_VEC = _CHUNK // 16
_MAXK = 13

_PHI_N = 200
_PHI2_N = 100
_PHI_PAD = 208
_PHI2_PAD = 112
_HIDDEN = 3
_SCALE1 = float(_PHI_N - 1)
_SCALE2 = float(_PHI2_N - 1) / 6.0


def _sc_body(x_hbm, phi_hbm, big_hbm, par_hbm, out_hbm,
             xb0, xb1, ob0, ob1, phib, bigb, parb,
             isem0, isem1, osem0, osem1):
    nchunks = x_hbm.shape[0] // _CHUNK
    wid = lax.axis_index("s") * 2 + lax.axis_index("c")
    nk = jnp.where(wid < nchunks % _NW, _MAXK, _MAXK - 1)

    pltpu.sync_copy(phi_hbm, phib)
    pltpu.sync_copy(big_hbm, bigb)
    pltpu.sync_copy(par_hbm, parb)
    eta_v = parb[pl.ds(0, 16)]
    lam_v = parb[pl.ds(16, 16)]
    shift = [eta_v * (_SCALE1 * q) for q in range(_HIDDEN)]
    lam2 = lam_v * _SCALE2

    xbufs, obufs = (xb0, xb1), (ob0, ob1)
    isems, osems = (isem0, isem1), (osem0, osem1)

    def start_in(k, b):
        off = (wid + _NW * k) * _CHUNK
        pltpu.async_copy(x_hbm.at[pl.ds(off, _CHUNK)], xbufs[b], isems[b])

    def wait_in(b):
        pltpu.make_async_copy(
            x_hbm.at[pl.ds(0, _CHUNK)], xbufs[b], isems[b]).wait()

    def start_out(k, b):
        off = (wid + _NW * k) * _CHUNK
        pltpu.async_copy(obufs[b], out_hbm.at[pl.ds(off, _CHUNK)], osems[b])

    def wait_out(b):
        pltpu.make_async_copy(
            obufs[b], out_hbm.at[pl.ds(0, _CHUNK)], osems[b]).wait()

    def compute(b):
        xb, ob = xbufs[b], obufs[b]

        @plsc.parallel_loop(0, _VEC, unroll=5)
        def _vec(i):
            v = xb[pl.ds(i * 16, 16)]
            acc = None
            for q in range(_HIDDEN):
                f = jnp.clip(v * _SCALE1 + shift[q], 0.0, _SCALE1)
                ii = jnp.minimum(f.astype(jnp.int32), _PHI_N - 2)
                t = f - ii.astype(jnp.float32)
                c0 = plsc.load_gather(phib, [ii])
                c1 = plsc.load_gather(phib, [ii + 1])
                phi = c0 + t * (c1 - c0)
                f2 = jnp.clip(phi * lam2 + (_SCALE2 * (q + 3.0)),
                              0.0, 6.0 * _SCALE2)
                jj = jnp.minimum(f2.astype(jnp.int32), _PHI2_N - 2)
                t2 = f2 - jj.astype(jnp.float32)
                d0 = plsc.load_gather(bigb, [jj])
                d1 = plsc.load_gather(bigb, [jj + 1])
                r = d0 + t2 * (d1 - d0)
                acc = r if acc is None else acc + r
            ob[pl.ds(i * 16, 16)] = acc

    start_in(0, 0)
    start_in(1, 1)

    @pl.loop(0, _MAXK - 1, step=2)
    def _pair(k):
        for b in range(2):
            kk = k + b
            wait_in(b)

            @pl.when(kk >= 2)
            def _drain():
                wait_out(b)

            compute(b)
            start_out(kk, b)

            @pl.when(kk + 2 < nk)
            def _next():
                start_in(kk + 2, b)

    @pl.when(nk == _MAXK)
    def _tail():
        wait_in(0)
        wait_out(0)
        compute(0)
        start_out(_MAXK - 1, 0)

    wait_out(0)
    wait_out(1)


def _make_sc_kernel(n):
    mesh = plsc.VectorSubcoreMesh(core_axis_name="c", subcore_axis_name="s")
    return pl.kernel(
        _sc_body,
        mesh=mesh,
        compiler_params=pltpu.CompilerParams(needs_layout_passes=False),
        out_type=jax.ShapeDtypeStruct((n,), jnp.float32),
        scratch_types=[
            pltpu.VMEM((_CHUNK,), jnp.float32),
            pltpu.VMEM((_CHUNK,), jnp.float32),
            pltpu.VMEM((_CHUNK,), jnp.float32),
            pltpu.VMEM((_CHUNK,), jnp.float32),
            pltpu.VMEM((_PHI_PAD,), jnp.float32),
            pltpu.VMEM((_PHI2_PAD,), jnp.float32),
            pltpu.VMEM((32,), jnp.float32),
            pltpu.SemaphoreType.DMA,
            pltpu.SemaphoreType.DMA,
            pltpu.SemaphoreType.DMA,
            pltpu.SemaphoreType.DMA,
        ],
    )


def kernel(x, phi_coeffs, Phi_coeffs, lambdas, eta):
    n = x.shape[0]
    xf = x.reshape(n)
    phi_p = jnp.zeros((_PHI_PAD,), jnp.float32).at[:_PHI_N].set(phi_coeffs)
    big_p = jnp.zeros((_PHI2_PAD,), jnp.float32).at[:_PHI2_N].set(Phi_coeffs)
    par = jnp.concatenate([
        jnp.full((16,), eta, jnp.float32),
        jnp.full((16,), lambdas[0], jnp.float32),
    ])
    out = _make_sc_kernel(n)(xf, phi_p, big_p, par)
    return out.reshape(n, 1)
